# Initial kernel scaffold; baseline (speedup 1.0000x reference)
import jax
import jax.numpy as jnp
from jax.experimental import pallas as pl


def kernel(x, edges, adj, W, b, W1, b1, W2, b2):
    raise NotImplementedError("write your pallas kernel here")


# speedup vs baseline: 3.7226x; 3.7226x over previous
import functools

import jax
import jax.numpy as jnp
from jax import lax
from jax.experimental import pallas as pl
from jax.experimental.pallas import tpu as pltpu
from jax.experimental.pallas import tpu_sc as plsc

NC = 2
NS = 16
LANES = 16


def _xw_tc(x, Wh):
  N, D = x.shape
  HC = Wh.shape[2]
  BM = 1000

  def body(x_ref, w_ref, o_ref):
    o_ref[...] = jnp.dot(x_ref[...], w_ref[0],
                         preferred_element_type=jnp.float32)

  return pl.pallas_call(
      body,
      grid=(2, N // BM),
      in_specs=[
          pl.BlockSpec((BM, D), lambda c, m: (m, 0)),
          pl.BlockSpec((1, D, HC), lambda c, m: (c, 0, 0)),
      ],
      out_specs=pl.BlockSpec((BM, HC), lambda c, m: (c * (N // BM) + m, 0)),
      out_shape=jax.ShapeDtypeStruct((2 * N, HC), jnp.float32),
  )(x, Wh)


def _mlp_tc(z0, z1, W1a, W1b, b1r, w2r, b2r):
  Qp, HC = z0.shape
  H = W1a.shape[1]
  BQ = 2048

  def body(z0_ref, z1_ref, w1a_ref, w1b_ref, b1_ref, w2_ref, b2_ref, o_ref):
    t = jnp.dot(z0_ref[...], w1a_ref[...], preferred_element_type=jnp.float32)
    t = t + jnp.dot(z1_ref[...], w1b_ref[...],
                    preferred_element_type=jnp.float32)
    t = jnp.maximum(t + b1_ref[...], 0.0)
    s = jnp.sum(t * w2_ref[...], axis=1) + b2_ref[0, 0]
    o_ref[...] = jax.nn.sigmoid(s)

  return pl.pallas_call(
      body,
      grid=(Qp // BQ,),
      in_specs=[
          pl.BlockSpec((BQ, HC), lambda i: (i, 0)),
          pl.BlockSpec((BQ, HC), lambda i: (i, 0)),
          pl.BlockSpec((HC, H), lambda i: (0, 0)),
          pl.BlockSpec((HC, H), lambda i: (0, 0)),
          pl.BlockSpec((1, H), lambda i: (0, 0)),
          pl.BlockSpec((1, H), lambda i: (0, 0)),
          pl.BlockSpec((1, 1), lambda i: (0, 0)),
      ],
      out_specs=pl.BlockSpec((BQ,), lambda i: (i,)),
      out_shape=jax.ShapeDtypeStruct((Qp,), jnp.float32),
  )(z0, z1, W1a, W1b, b1r, w2r, b2r)


def _sc_mega(xw2, srcb, dstb, e0b, e1b, b2v, N, HC, TB, TQ, Q_pad, AGG_R):
  mesh = plsc.VectorSubcoreMesh(core_axis_name="c", subcore_axis_name="s")
  rows_per_tile = AGG_R // NS

  @functools.partial(
      pl.kernel,
      out_type=jax.ShapeDtypeStruct((2, Q_pad, HC), jnp.float32),
      mesh=mesh,
      compiler_params=pltpu.CompilerParams(use_tc_tiling_on_sc=False),
      scratch_types=[
          pltpu.VMEM((TB, 128), jnp.int32),
          pltpu.VMEM((TB, 128), jnp.int32),
          pltpu.VMEM((128, HC), jnp.float32),
          pltpu.VMEM((TQ, 128), jnp.int32),
          pltpu.VMEM((TQ, 128), jnp.int32),
          pltpu.VMEM((128, HC), jnp.float32),
          pltpu.VMEM((128, HC), jnp.float32),
          pltpu.VMEM((128, HC), jnp.float32),
          pltpu.VMEM((HC,), jnp.float32),
          pltpu.VMEM_SHARED((AGG_R, HC), jnp.float32),
          pltpu.SemaphoreType.DMA,
          pltpu.SemaphoreType.DMA,
      ],
  )
  def k(xw2_h, srcb_h, dstb_h, e0b_h, e1b_h, b2v_h, zout_h,
        sidx, didx, rows, e0i, e1i, g0, g1, zb, bv, agg, sem, sem2):
    cid = lax.axis_index("c")
    tid = lax.axis_index("s")

    def zrow(r, carry):
      for k4 in range(HC // LANES):
        rows[r, pl.ds(k4 * LANES, LANES)] = jnp.zeros((LANES,), jnp.float32)
      return carry

    lax.fori_loop(0, 128, zrow, 0)
    for kk in range(rows_per_tile // 128):
      pltpu.sync_copy(rows, agg.at[pl.ds(tid * rows_per_tile + kk * 128, 128)])
    plsc.subcore_barrier()

    pltpu.sync_copy(srcb_h.at[cid, tid], sidx)
    pltpu.sync_copy(dstb_h.at[tid], didx)

    def edge_chunk(j, carry):
      pltpu.async_copy(xw2_h.at[sidx.at[j]], rows, sem).wait()
      pltpu.sync_copy(rows, agg.at[didx.at[j]], add=True)
      return carry

    lax.fori_loop(0, TB, edge_chunk, 0)
    plsc.subcore_barrier()

    pltpu.sync_copy(b2v_h.at[cid], bv)
    pltpu.sync_copy(e0b_h.at[tid], e0i)
    pltpu.sync_copy(e1b_h.at[tid], e1i)
    bks = [bv[pl.ds(k4 * LANES, LANES)] for k4 in range(HC // LANES)]

    def query_chunk(q, carry):
      pltpu.async_copy(agg.at[e0i.at[q]], g0, sem).wait()
      pltpu.async_copy(agg.at[e1i.at[q]], g1, sem2).wait()

      def prod(r, c2):
        for k4 in range(HC // LANES):
          sl = pl.ds(k4 * LANES, LANES)
          a0 = jnp.maximum(g0[r, sl] + bks[k4], 0.0)
          a1 = jnp.maximum(g1[r, sl] + bks[k4], 0.0)
          zb[r, sl] = a0 * a1
        return c2

      lax.fori_loop(0, 128, prod, 0)
      pltpu.sync_copy(zb, zout_h.at[cid, pl.ds((tid * TQ + q) * 128, 128)])
      return carry

    lax.fori_loop(0, TQ, query_chunk, 0)

  return k(xw2, srcb, dstb, e0b, e1b, b2v)


def kernel(x, edges, adj, W, b, W1, b1, W2, b2):
  N, D = x.shape
  H = W.shape[1]
  HC = H // 2
  E = adj.shape[1]
  Q = edges.shape[1]

  TB = -(-E // (NS * 128))
  E_pad = NS * TB * 128
  TQ = -(-Q // (NS * 128))
  Q_pad = NS * TQ * 128
  AGG_R = NS * (-(-(N + 1) // (NS * 128)) * 128)

  src = adj[0].astype(jnp.int32)
  dst = adj[1].astype(jnp.int32)
  e0 = edges[0].astype(jnp.int32)
  e1 = edges[1].astype(jnp.int32)

  src_p = jnp.concatenate([src, jnp.zeros((E_pad - E,), jnp.int32)])
  dst_p = jnp.concatenate([dst, jnp.full((E_pad - E,), N, jnp.int32)])
  srcb = jnp.stack([src_p, src_p + N]).reshape(2, NS, TB, 128)
  dstb = dst_p.reshape(NS, TB, 128)
  e0b = jnp.concatenate([e0, jnp.zeros((Q_pad - Q,), jnp.int32)])
  e0b = e0b.reshape(NS, TQ, 128)
  e1b = jnp.concatenate([e1, jnp.zeros((Q_pad - Q,), jnp.int32)])
  e1b = e1b.reshape(NS, TQ, 128)

  xw2 = _xw_tc(x, jnp.stack([W[:, :HC], W[:, HC:]]))
  zout = _sc_mega(xw2, srcb, dstb, e0b, e1b, b.reshape(2, HC),
                  N, HC, TB, TQ, Q_pad, AGG_R)
  out = _mlp_tc(zout[0], zout[1], W1[:HC], W1[HC:],
                b1.reshape(1, H), W2.reshape(1, H), b2.reshape(1, 1))
  return out[:Q]
